# baseline (speedup 1.0000x reference)
import functools

import jax
import jax.numpy as jnp
from jax import lax
from jax.experimental import pallas as pl
from jax.experimental.pallas import tpu as pltpu
from jax.experimental.pallas import tpu_sc as plsc

_B, _S, _D = 4, 8192, 1024
_NC, _NS = 2, 16
_NW = _NC * _NS
_SPW = _S // _NW
_CH = 4
_NCHUNK = _SPW // _CH
_NSET = 6
_PD = 3
_CU = 8

_mesh = plsc.VectorSubcoreMesh(core_axis_name="c", subcore_axis_name="s")


@functools.partial(
    pl.kernel,
    mesh=_mesh,
    out_type=jax.ShapeDtypeStruct((_B * _S, _D), jnp.float32),
    scratch_types=(
        [pltpu.VMEM((_CH, _D), jnp.float32) for _ in range(_NSET * _B)]
        + [pltpu.VMEM((_CH, _D), jnp.float32) for _ in range(_NSET)]
        + [pltpu.SemaphoreType.DMA for _ in range(_NSET * _B + _NSET)]
    ),
)
def _sc_add(x_hbm, pe_hbm, out_hbm, *scratch):
    x_bufs = [list(scratch[p * _B:(p + 1) * _B]) for p in range(_NSET)]
    pe_bufs = list(scratch[_NSET * _B:_NSET * _B + _NSET])
    sems = scratch[_NSET * _B + _NSET:]
    x_sems = [list(sems[p * _B:(p + 1) * _B]) for p in range(_NSET)]
    pe_sems = list(sems[_NSET * _B:_NSET * _B + _NSET])

    wid = lax.axis_index("s") * _NC + lax.axis_index("c")
    s_base = wid * _SPW

    def pe_row(c):
        return s_base + c * _CH

    def x_row(c, b):
        return b * _S + pe_row(c)

    def issue_loads(c, p):
        pltpu.async_copy(
            pe_hbm.at[pl.ds(pe_row(c), _CH)], pe_bufs[p], pe_sems[p])
        for b in range(_B):
            pltpu.async_copy(
                x_hbm.at[pl.ds(x_row(c, b), _CH)], x_bufs[p][b], x_sems[p][b])

    def wait_loads(p):
        pltpu.make_async_copy(
            pe_hbm.at[pl.ds(0, _CH)], pe_bufs[p], pe_sems[p]).wait()
        for b in range(_B):
            pltpu.make_async_copy(
                x_hbm.at[pl.ds(0, _CH)], x_bufs[p][b], x_sems[p][b]).wait()

    def issue_stores(c, p):
        for b in range(_B):
            pltpu.async_copy(
                x_bufs[p][b], out_hbm.at[pl.ds(x_row(c, b), _CH)],
                x_sems[p][b])

    def wait_stores(p):
        for b in range(_B):
            pltpu.make_async_copy(
                x_bufs[p][b], out_hbm.at[pl.ds(0, _CH)], x_sems[p][b]).wait()

    def compute(p):
        for r in range(_CH):
            def col_body(j, carry, r=r, p=p):
                base = j * 16 * _CU
                for u in range(_CU):
                    sl = pl.ds(base + u * 16, 16)
                    v = pe_bufs[p][r, sl]
                    for b in range(_B):
                        x_bufs[p][b][r, sl] = x_bufs[p][b][r, sl] + v
                return carry

            lax.fori_loop(0, _D // (16 * _CU), col_body, 0)

    for c in range(_PD):
        issue_loads(c, c)

    for j in range(_NSET):
        p, p2 = j, (j + _PD) % _NSET
        if j < _NSET - _PD:
            issue_loads(j + _PD, p2)
        else:
            wait_stores(p2)
            issue_loads(j + _PD, p2)
        wait_loads(p)
        compute(p)
        issue_stores(j, p)

    _NROT = _NCHUNK // _NSET
    _TAIL = _NCHUNK - _NROT * _NSET

    def rotation(cp, carry):
        for j in range(_NSET):
            c = cp * _NSET + j
            p, p2 = j, (j + _PD) % _NSET
            wait_stores(p2)
            issue_loads(c + _PD, p2)
            wait_loads(p)
            compute(p)
            issue_stores(c, p)
        return carry

    lax.fori_loop(1, _NROT, rotation, 0)

    for j in range(_TAIL):
        c = _NROT * _NSET + j
        p, p2 = j, (j + _PD) % _NSET
        if c + _PD < _NCHUNK:
            wait_stores(p2)
            issue_loads(c + _PD, p2)
        wait_loads(p)
        compute(p)
        issue_stores(c, p)

    for k in range(_NSET):
        wait_stores((_NCHUNK - _NSET + k) % _NSET)


def kernel(x, pe_table):
    B, S, D = x.shape
    out = _sc_add(x.reshape(B * S, D), pe_table[:S])
    return out.reshape(B, S, D)


# speedup vs baseline: 1.0532x; 1.0532x over previous
import functools

import jax
import jax.numpy as jnp
from jax import lax
from jax.experimental import pallas as pl
from jax.experimental.pallas import tpu as pltpu
from jax.experimental.pallas import tpu_sc as plsc

_B, _S, _D = 4, 8192, 1024
_NC, _NS = 2, 16
_NW = _NC * _NS
_SPW = _S // _NW
_CH = 4
_NCHUNK = _SPW // _CH
_NSET = 4
_CU = 8

_mesh = plsc.VectorSubcoreMesh(core_axis_name="c", subcore_axis_name="s")


@functools.partial(
    pl.kernel,
    mesh=_mesh,
    out_type=jax.ShapeDtypeStruct((_B, _S, _D), jnp.float32),
    scratch_types=(
        [pltpu.VMEM((_B, _CH, _D), jnp.float32) for _ in range(_NSET)]
        + [pltpu.VMEM((_CH, _D), jnp.float32) for _ in range(_NSET)]
        + [pltpu.SemaphoreType.DMA for _ in range(2 * _NSET)]
    ),
)
def _sc_add(x_hbm, pe_hbm, out_hbm, *scratch):
    x_bufs = list(scratch[:_NSET])
    pe_bufs = list(scratch[_NSET:2 * _NSET])
    x_sems = list(scratch[2 * _NSET:3 * _NSET])
    pe_sems = list(scratch[3 * _NSET:])

    wid = lax.axis_index("s") * _NC + lax.axis_index("c")
    s_base = wid * _SPW

    def row0(c):
        return s_base + c * _CH

    def issue_loads(c, p):
        pltpu.async_copy(
            pe_hbm.at[pl.ds(row0(c), _CH)], pe_bufs[p], pe_sems[p])
        pltpu.async_copy(
            x_hbm.at[:, pl.ds(row0(c), _CH)], x_bufs[p], x_sems[p])

    def wait_loads(p):
        pltpu.make_async_copy(
            pe_hbm.at[pl.ds(0, _CH)], pe_bufs[p], pe_sems[p]).wait()
        pltpu.make_async_copy(
            x_hbm.at[:, pl.ds(0, _CH)], x_bufs[p], x_sems[p]).wait()

    def issue_stores(c, p):
        pltpu.async_copy(
            x_bufs[p], out_hbm.at[:, pl.ds(row0(c), _CH)], x_sems[p])

    def wait_stores(p):
        pltpu.make_async_copy(
            x_bufs[p], out_hbm.at[:, pl.ds(0, _CH)], x_sems[p]).wait()

    def compute(p):
        for r in range(_CH):
            def col_body(j, carry, r=r, p=p):
                base = j * 16 * _CU
                for u in range(_CU):
                    sl = pl.ds(base + u * 16, 16)
                    v = pe_bufs[p][r, sl]
                    for b in range(_B):
                        x_bufs[p][b, r, sl] = x_bufs[p][b, r, sl] + v
                return carry

            lax.fori_loop(0, _D // (16 * _CU), col_body, 0)

    issue_loads(0, 0)
    issue_loads(1, 1)

    for j in range(_NSET):
        p, p2 = j, (j + 2) % _NSET
        if j < 2:
            issue_loads(j + 2, p2)
        else:
            wait_stores(p2)
            issue_loads(j + 2, p2)
        wait_loads(p)
        compute(p)
        issue_stores(j, p)

    def rotation(cp, carry):
        for j in range(_NSET):
            c = cp * _NSET + j
            p, p2 = j, (j + 2) % _NSET
            wait_stores(p2)
            issue_loads(c + 2, p2)
            wait_loads(p)
            compute(p)
            issue_stores(c, p)
        return carry

    lax.fori_loop(1, _NCHUNK // _NSET - 1, rotation, 0)

    for j in range(_NSET):
        c = (_NCHUNK - _NSET) + j
        p, p2 = j, (j + 2) % _NSET
        if c + 2 < _NCHUNK:
            wait_stores(p2)
            issue_loads(c + 2, p2)
        wait_loads(p)
        compute(p)
        issue_stores(c, p)

    for p in range(_NSET):
        wait_stores(p)


def kernel(x, pe_table):
    S = x.shape[1]
    return _sc_add(x, pe_table[:S])
